# baseline (speedup 1.0000x reference)
import functools

import jax
import jax.numpy as jnp
from jax import lax
from jax.experimental import pallas as pl
from jax.experimental.pallas import tpu as pltpu
from jax.experimental.pallas import tpu_sc as plsc

E = 320000
N_NODES = 10000
D = 128
L = 16
NC = 2
NS = 16
NW = NC * NS
EPW = E // NW
C = 32
NCHUNK = EPW // C
TAIL = EPW - NCHUNK * C
NPAIR = NCHUNK // 2

_mesh = plsc.VectorSubcoreMesh(core_axis_name="c", subcore_axis_name="s")


@functools.partial(
    pl.kernel,
    out_type=jax.ShapeDtypeStruct((E,), jnp.float32),
    mesh=_mesh,
    scratch_types=[
        pltpu.VMEM((2 * EPW,), jnp.int32),
        pltpu.VMEM((2 * C, D), jnp.float32),
        pltpu.VMEM((2 * C, D), jnp.float32),
        pltpu.VMEM((EPW,), jnp.float32),
        pltpu.VMEM((L * L,), jnp.float32),
        pltpu.SemaphoreType.DMA,
        pltpu.SemaphoreType.DMA,
    ],
    compiler_params=pltpu.CompilerParams(needs_layout_passes=False),
)
def _decode(z_hbm, idx_hbm, out_hbm,
            idxv, rows_a, rows_b, oval, tstage, sem_a, sem_b):
    wid = lax.axis_index("s") * NC + lax.axis_index("c")
    base = wid * EPW
    pltpu.sync_copy(idx_hbm.at[pl.ds(wid * 2 * EPW, 2 * EPW)], idxv)

    def start(ci, rows, sem):
        pltpu.async_copy(z_hbm.at[idxv.at[pl.ds(ci * 2 * C, 2 * C)]], rows, sem)

    def wait(rows, sem):
        pltpu.make_async_copy(z_hbm.at[idxv.at[pl.ds(0, 2 * C)]], rows, sem).wait()

    def compute(ci, rows):
        def group_body(g, carry2):
            lanes = lax.iota(jnp.int32, L)
            for e in range(L):
                row = g * L + e
                acc = rows[row, pl.ds(0, L)] * rows[C + row, pl.ds(0, L)]
                for k in range(1, D // L):
                    acc = acc + (rows[row, pl.ds(k * L, L)]
                                 * rows[C + row, pl.ds(k * L, L)])
                plsc.store_scatter(tstage, [lanes * L + e], acc)
            dots = tstage[pl.ds(0, L)]
            for r in range(1, L):
                dots = dots + tstage[pl.ds(r * L, L)]
            oval[pl.ds(ci * C + g * L, L)] = 1.0 / (1.0 + jnp.exp(-dots))
            return carry2

        lax.fori_loop(0, C // L, group_body, 0)

    start(0, rows_a, sem_a)

    def pair_body(p, carry):
        c0 = 2 * p
        start(c0 + 1, rows_b, sem_b)
        wait(rows_a, sem_a)
        compute(c0, rows_a)

        @pl.when(c0 + 2 < NCHUNK)
        def _():
            start(c0 + 2, rows_a, sem_a)

        wait(rows_b, sem_b)
        compute(c0 + 1, rows_b)
        return carry

    lax.fori_loop(0, NPAIR, pair_body, 0)

    pltpu.async_copy(
        z_hbm.at[idxv.at[pl.ds(NCHUNK * 2 * C, 2 * TAIL)]],
        rows_a.at[pl.ds(0, 2 * TAIL)], sem_a)
    pltpu.make_async_copy(
        z_hbm.at[idxv.at[pl.ds(0, 2 * TAIL)]],
        rows_a.at[pl.ds(0, 2 * TAIL)], sem_a).wait()
    lanes = lax.iota(jnp.int32, L)
    acc = rows_a[0, pl.ds(0, L)] * rows_a[TAIL, pl.ds(0, L)]
    for e in range(L):
        if e:
            acc = rows_a[e, pl.ds(0, L)] * rows_a[TAIL + e, pl.ds(0, L)]
        for k in range(1, D // L):
            acc = acc + (rows_a[e, pl.ds(k * L, L)]
                         * rows_a[TAIL + e, pl.ds(k * L, L)])
        plsc.store_scatter(tstage, [lanes * L + e], acc)
    dots = tstage[pl.ds(0, L)]
    for r in range(1, L):
        dots = dots + tstage[pl.ds(r * L, L)]
    oval[pl.ds(NCHUNK * C, L)] = 1.0 / (1.0 + jnp.exp(-dots))

    pltpu.sync_copy(oval, out_hbm.at[pl.ds(base, EPW)])


def kernel(z, edge_index):
    ei = edge_index.astype(jnp.int32)
    s = ei[0].reshape(NW, EPW)
    d = ei[1].reshape(NW, EPW)
    main = jnp.stack(
        [s[:, :NCHUNK * C].reshape(NW, NCHUNK, C),
         d[:, :NCHUNK * C].reshape(NW, NCHUNK, C)], axis=2)
    tail = jnp.stack([s[:, NCHUNK * C:], d[:, NCHUNK * C:]], axis=1)
    idxall = jnp.concatenate(
        [main.reshape(NW, -1), tail.reshape(NW, -1)], axis=1).reshape(-1)
    return _decode(z, idxall)


# speedup vs baseline: 1.5855x; 1.5855x over previous
import functools

import jax
import jax.numpy as jnp
from jax import lax
from jax.experimental import pallas as pl
from jax.experimental.pallas import tpu as pltpu
from jax.experimental.pallas import tpu_sc as plsc

E = 320000
N_NODES = 10000
D = 128
L = 16
NC = 2
NS = 16
NW = NC * NS
EPW = E // NW
C = 32
NCHUNK = EPW // C
TAIL = EPW - NCHUNK * C
NPAIR = NCHUNK // 2

_mesh = plsc.VectorSubcoreMesh(core_axis_name="c", subcore_axis_name="s")


@functools.partial(
    pl.kernel,
    out_type=jax.ShapeDtypeStruct((E,), jnp.float32),
    mesh=_mesh,
    scratch_types=[
        pltpu.VMEM_SHARED((N_NODES, D), jnp.float32),
        pltpu.VMEM((EPW,), jnp.int32),
        pltpu.VMEM((EPW,), jnp.int32),
        pltpu.VMEM((C, D), jnp.float32),
        pltpu.VMEM((C, D), jnp.float32),
        pltpu.VMEM((C, D), jnp.float32),
        pltpu.VMEM((C, D), jnp.float32),
        pltpu.VMEM((EPW,), jnp.float32),
        pltpu.VMEM((L * L,), jnp.float32),
        pltpu.SemaphoreType.DMA,
        pltpu.SemaphoreType.DMA,
        pltpu.SemaphoreType.DMA,
        pltpu.SemaphoreType.DMA,
    ],
    compiler_params=pltpu.CompilerParams(needs_layout_passes=False),
)
def _decode(z_hbm, src_hbm, dst_hbm, out_hbm,
            z_sp, sidx, didx, srows_a, drows_a, srows_b, drows_b, oval, tstage,
            sem_sa, sem_da, sem_sb, sem_db):
    wid = lax.axis_index("s") * NC + lax.axis_index("c")
    sid = lax.axis_index("s")
    base = wid * EPW
    rps = (N_NODES // NS) // 8 * 8
    pltpu.sync_copy(z_hbm.at[pl.ds(sid * rps, rps)],
                    z_sp.at[pl.ds(sid * rps, rps)])
    tail = N_NODES - NS * rps

    @pl.when(sid == 0)
    def _copy_tail():
        pltpu.sync_copy(z_hbm.at[pl.ds(NS * rps, tail)],
                        z_sp.at[pl.ds(NS * rps, tail)])
    pltpu.sync_copy(src_hbm.at[pl.ds(base, EPW)], sidx)
    pltpu.sync_copy(dst_hbm.at[pl.ds(base, EPW)], didx)
    plsc.subcore_barrier()

    def start(ci, srows, drows, sem_s, sem_d):
        pltpu.async_copy(z_sp.at[sidx.at[pl.ds(ci * C, C)]], srows, sem_s)
        pltpu.async_copy(z_sp.at[didx.at[pl.ds(ci * C, C)]], drows, sem_d)

    def wait(srows, drows, sem_s, sem_d):
        pltpu.make_async_copy(z_sp.at[sidx.at[pl.ds(0, C)]], srows, sem_s).wait()
        pltpu.make_async_copy(z_sp.at[didx.at[pl.ds(0, C)]], drows, sem_d).wait()

    def compute(ci, srows, drows):
        def group_body(g, carry2):
            lanes = lax.iota(jnp.int32, L)
            for e in range(L):
                row = g * L + e
                acc = srows[row, pl.ds(0, L)] * drows[row, pl.ds(0, L)]
                for k in range(1, D // L):
                    acc = acc + (srows[row, pl.ds(k * L, L)]
                                 * drows[row, pl.ds(k * L, L)])
                plsc.store_scatter(tstage, [lanes * L + e], acc)
            dots = tstage[pl.ds(0, L)]
            for r in range(1, L):
                dots = dots + tstage[pl.ds(r * L, L)]
            oval[pl.ds(ci * C + g * L, L)] = 1.0 / (1.0 + jnp.exp(-dots))
            return carry2

        lax.fori_loop(0, C // L, group_body, 0)

    start(0, srows_a, drows_a, sem_sa, sem_da)

    def pair_body(p, carry):
        c0 = 2 * p
        start(c0 + 1, srows_b, drows_b, sem_sb, sem_db)
        wait(srows_a, drows_a, sem_sa, sem_da)
        compute(c0, srows_a, drows_a)

        @pl.when(c0 + 2 < NCHUNK)
        def _():
            start(c0 + 2, srows_a, drows_a, sem_sa, sem_da)

        wait(srows_b, drows_b, sem_sb, sem_db)
        compute(c0 + 1, srows_b, drows_b)
        return carry

    lax.fori_loop(0, NPAIR, pair_body, 0)

    pltpu.async_copy(z_sp.at[sidx.at[pl.ds(NCHUNK * C, TAIL)]],
                     srows_a.at[pl.ds(0, TAIL)], sem_sa)
    pltpu.async_copy(z_sp.at[didx.at[pl.ds(NCHUNK * C, TAIL)]],
                     drows_a.at[pl.ds(0, TAIL)], sem_da)
    pltpu.make_async_copy(z_sp.at[sidx.at[pl.ds(0, TAIL)]],
                          srows_a.at[pl.ds(0, TAIL)], sem_sa).wait()
    pltpu.make_async_copy(z_sp.at[didx.at[pl.ds(0, TAIL)]],
                          drows_a.at[pl.ds(0, TAIL)], sem_da).wait()
    lanes = lax.iota(jnp.int32, L)
    for e in range(L):
        acc = srows_a[e, pl.ds(0, L)] * drows_a[e, pl.ds(0, L)]
        for k in range(1, D // L):
            acc = acc + (srows_a[e, pl.ds(k * L, L)]
                         * drows_a[e, pl.ds(k * L, L)])
        plsc.store_scatter(tstage, [lanes * L + e], acc)
    dots = tstage[pl.ds(0, L)]
    for r in range(1, L):
        dots = dots + tstage[pl.ds(r * L, L)]
    oval[pl.ds(NCHUNK * C, L)] = 1.0 / (1.0 + jnp.exp(-dots))

    pltpu.sync_copy(oval, out_hbm.at[pl.ds(base, EPW)])


def kernel(z, edge_index):
    ei = edge_index.astype(jnp.int32)
    return _decode(z, ei[0], ei[1])
